# baseline (speedup 1.0000x reference)
import functools

import jax
import jax.numpy as jnp
from jax import lax
from jax.experimental import pallas as pl
from jax.experimental.pallas import tpu as pltpu
from jax.experimental.pallas import tpu_sc as plsc

FEATURE = 1024
QUEUE = 65536
BATCH = 4096
NC, NS = 2, 16
NW = NC * NS
WROWS = FEATURE // NW
CC = 1024
NKC = BATCH // CC
NCH = QUEUE // CC


def _chunk_in(tkeys_ref, queue_ref, r0, i, buf, sem):
    src = tkeys_ref if i < NKC else queue_ref
    return pltpu.make_async_copy(
        src.at[pl.ds(r0, WROWS), pl.ds(i * CC, CC)], buf, sem)


def _chunk_out(out_ref, r0, i, buf, sem):
    return pltpu.make_async_copy(
        buf, out_ref.at[pl.ds(r0, WROWS), pl.ds(i * CC, CC)], sem)


def _sc_body(tkeys_ref, queue_ref, out_ref,
             buf0, buf1, isem0, isem1, osem0, osem1):
    wid = lax.axis_index("s") * NC + lax.axis_index("c")
    r0 = wid * WROWS

    bufs = (buf0, buf1)
    isems = (isem0, isem1)
    osems = (osem0, osem1)

    _chunk_in(tkeys_ref, queue_ref, r0, 0, buf0, isem0).start()
    _chunk_in(tkeys_ref, queue_ref, r0, 1, buf1, isem1).start()
    for i in range(NCH):
        s = i % 2
        _chunk_in(tkeys_ref, queue_ref, r0, i, bufs[s], isems[s]).wait()
        if i >= 2:
            _chunk_out(out_ref, r0, i - 2, bufs[s], osems[s]).wait()
        _chunk_out(out_ref, r0, i, bufs[s], osems[s]).start()
        if i + 2 < NCH:
            _chunk_in(tkeys_ref, queue_ref, r0, i + 2, bufs[s], isems[s]).start()
    _chunk_out(out_ref, r0, NCH - 2, bufs[(NCH - 2) % 2], osems[(NCH - 2) % 2]).wait()
    _chunk_out(out_ref, r0, NCH - 1, bufs[(NCH - 1) % 2], osems[(NCH - 1) % 2]).wait()


def _t_body(keys_ref, tk_ref):
    tk_ref[...] = keys_ref[...].T


def _transpose_tc(keys):
    return pl.pallas_call(
        _t_body,
        grid=(BATCH // CC,),
        in_specs=[pl.BlockSpec((CC, FEATURE), lambda j: (j, 0))],
        out_specs=pl.BlockSpec((FEATURE, CC), lambda j: (0, j)),
        out_shape=jax.ShapeDtypeStruct((FEATURE, BATCH), jnp.float32),
    )(keys)


def kernel(keys, queue):
    tkeys = _transpose_tc(keys)
    mesh = plsc.VectorSubcoreMesh(core_axis_name="c", subcore_axis_name="s")
    k = functools.partial(
        pl.kernel,
        out_type=jax.ShapeDtypeStruct((FEATURE, QUEUE), jnp.float32),
        mesh=mesh,
        scratch_types=[
            pltpu.VMEM((WROWS, CC), jnp.float32),
            pltpu.VMEM((WROWS, CC), jnp.float32),
            pltpu.SemaphoreType.DMA,
            pltpu.SemaphoreType.DMA,
            pltpu.SemaphoreType.DMA,
            pltpu.SemaphoreType.DMA,
        ],
    )(_sc_body)
    return k(tkeys, queue)
